# baseline (speedup 1.0000x reference)
import functools

import jax
import jax.numpy as jnp
from jax import lax
from jax.experimental import pallas as pl
from jax.experimental.pallas import tpu as pltpu
from jax.experimental.pallas import tpu_sc as plsc

B, P, H, W, D, C = 4, 8, 32, 32, 32, 16
NVOL = B * P
VPV = H * W * D
CHUNK = 128
NCHUNK = VPV // CHUNK
NGROUP = CHUNK // 16


NPRE = 4096


def _sc_resample(vol_hbm, thetab_hbm, out_hbm, tbl_hbm, theta_v, idx_b, w_b,
                 cbuf, outb, pre0, sem0, sem1):
  wid = lax.axis_index("s") * 2 + lax.axis_index("c")
  base_row = wid * VPV
  sems = (sem0, sem1)

  pltpu.sync_copy(thetab_hbm.at[wid], theta_v)

  for t in range(VPV // NPRE):
    b0 = base_row + t * NPRE
    pltpu.sync_copy(vol_hbm.at[pl.ds(b0, NPRE)], pre0)
    pltpu.sync_copy(pre0, tbl_hbm.at[pl.ds(b0 + 1, NPRE), pl.ds(0, C)])
    pltpu.sync_copy(pre0, tbl_hbm.at[pl.ds(b0, NPRE), pl.ds(C, C)])
  tv = [theta_v[m, :] for m in range(12)]

  iota_i = lax.iota(jnp.int32, 16)
  iota = iota_i.astype(jnp.float32)
  kvs = [iota, iota + 16.0]
  lane_ids = [jnp.full((16,), l, jnp.int32) for l in range(16)]

  def phase1(c, s):
    i_s = c // 8
    j0 = (c % 8) * 4
    iv = jnp.full((16,), i_s).astype(jnp.float32)

    for u in range(NGROUP):
      jv = jnp.full((16,), j0 + u // 2).astype(jnp.float32)
      kv = kvs[u % 2]

      fs = []
      ci = []
      zero = jnp.zeros((16,), jnp.float32)
      for r in range(3):
        t0, t1, t2, t3 = tv[4 * r:4 * r + 4]
        xs = t0 * jv + t1 * iv + t2 * kv + t3 + 2.0
        x0 = jnp.minimum(xs.astype(jnp.int32), 34)
        x0 = jnp.maximum(x0, 0)
        xd = xs - x0.astype(jnp.float32)
        xi0 = x0 - 2
        xi1 = x0 - 1
        f0 = jnp.where((xi0 >= 0) & (xi0 <= 31), 1.0 - xd, zero)
        f1 = jnp.where((xi1 >= 0) & (xi1 <= 31), xd, zero)
        fs.append((f0, f1))
        ci.append((xi0, xi1))

      ax = (jnp.clip(ci[0][0], 0, 31) * 1024 + base_row,
            jnp.clip(ci[0][1], 0, 31) * 1024 + base_row)
      by = (jnp.clip(ci[1][0], 0, 31) * 32, jnp.clip(ci[1][1], 0, 31) * 32)
      zi0, zi1 = ci[2]
      fz0, fz1 = fs[2]
      z_lo = jnp.clip(zi0, 0, 30)
      g0 = (jnp.where(zi0 == z_lo, fz0, zero)
            + jnp.where(zi1 == z_lo, fz1, zero))
      g1 = (jnp.where(zi0 == z_lo + 1, fz0, zero)
            + jnp.where(zi1 == z_lo + 1, fz1, zero))
      wx, wy = fs[0], fs[1]
      for xc in range(2):
        for yc in range(2):
          xy = (xc << 1) | yc
          rxy = ax[xc] + by[yc]
          wxy = wx[xc] * wy[yc]
          idx_b[s, xy, pl.ds(u * 16, 16)] = rxy + z_lo + 1
          w_b[s, 2 * xy, pl.ds(u * 16, 16)] = wxy * g0
          w_b[s, 2 * xy + 1, pl.ds(u * 16, 16)] = wxy * g1

    for xy in range(4):
      pltpu.async_copy(tbl_hbm.at[idx_b.at[s, xy]], cbuf.at[s, xy], sems[s])

  def combine(c, s):
    for xy in range(4):
      pltpu.make_async_copy(
          tbl_hbm.at[idx_b.at[s, xy]], cbuf.at[s, xy], sems[s]).wait()

    def group_body(g, _):
      wv = [w_b[s, cr, pl.ds(g * 16, 16)] for cr in range(8)]
      for l in range(16):
        v = g * 16 + l
        terms = []
        for cr in range(8):
          wb = wv[cr].at[lane_ids[l]].get(mode="promise_in_bounds")
          terms.append(wb * cbuf[s, cr // 2, v, pl.ds((cr % 2) * 16, 16)])
        t01 = terms[0] + terms[1]
        t23 = terms[2] + terms[3]
        t45 = terms[4] + terms[5]
        t67 = terms[6] + terms[7]
        outb[s, v, :] = (t01 + t23) + (t45 + t67)
      return 0

    lax.fori_loop(0, NGROUP, group_body, 0)
    pltpu.sync_copy(outb.at[s], out_hbm.at[pl.ds(base_row + c * CHUNK, CHUNK)])

  phase1(0, 0)

  def pair_body(c2, _):
    c = c2 * 2
    phase1(c + 1, 1)
    combine(c, 0)

    @pl.when(c2 < NCHUNK // 2 - 1)
    def _():
      phase1(c + 2, 0)

    combine(c + 1, 1)
    return 0

  lax.fori_loop(0, NCHUNK // 2, pair_body, 0)


@jax.jit
def kernel(input_fmap, theta):
  vol = input_fmap.reshape(NVOL * VPV, C)
  tb = lax.bitcast_convert_type(theta, jnp.uint32)
  tb = (tb + jnp.uint32(0x7FFF) + ((tb >> 16) & jnp.uint32(1)))
  tb = tb & jnp.uint32(0xFFFF0000)
  theta_r = lax.bitcast_convert_type(tb, jnp.float32)
  thetab = jnp.broadcast_to(theta_r.reshape(NVOL, 12, 1), (NVOL, 12, 16))

  mesh = plsc.VectorSubcoreMesh(core_axis_name="c", subcore_axis_name="s",
                                num_cores=2, num_subcores=16)
  run = functools.partial(
      pl.kernel,
      mesh=mesh,
      compiler_params=pltpu.CompilerParams(needs_layout_passes=False,
                                           use_tc_tiling_on_sc=False),
      out_type=[
          jax.ShapeDtypeStruct((NVOL * VPV, C), jnp.float32),
          jax.ShapeDtypeStruct((NVOL * VPV + 8, 2 * C), jnp.float32),
      ],
      scratch_types=[
          pltpu.VMEM((12, 16), jnp.float32),
          pltpu.VMEM((2, 4, CHUNK), jnp.int32),
          pltpu.VMEM((2, 8, CHUNK), jnp.float32),
          pltpu.VMEM((2, 4, CHUNK, 2 * C), jnp.float32),
          pltpu.VMEM((2, CHUNK, 16), jnp.float32),
          pltpu.VMEM((NPRE, C), jnp.float32),
          pltpu.SemaphoreType.DMA,
          pltpu.SemaphoreType.DMA,
      ],
  )(_sc_resample)
  out, _ = run(vol, thetab)
  return out.reshape(B, P, H, W, D, C)


# speedup vs baseline: 1.1041x; 1.1041x over previous
import functools

import jax
import jax.numpy as jnp
from jax import lax
from jax.experimental import pallas as pl
from jax.experimental.pallas import tpu as pltpu
from jax.experimental.pallas import tpu_sc as plsc

B, P, H, W, D, C = 4, 8, 32, 32, 32, 16
NVOL = B * P
VPV = H * W * D
CHUNK = 128
NCHUNK = VPV // CHUNK
NGROUP = CHUNK // 16


def _sc_resample(vol_hbm, thetab_hbm, out_hbm, theta_v, idx_b, w_b, cbuf,
                 outb, sem0, sem1):
  wid = lax.axis_index("s") * 2 + lax.axis_index("c")
  base_row = wid * VPV
  sems = (sem0, sem1)

  pltpu.sync_copy(thetab_hbm.at[wid], theta_v)
  tv = [theta_v[m, :] for m in range(12)]

  iota_i = lax.iota(jnp.int32, 16)
  iota = iota_i.astype(jnp.float32)
  kvs = [iota, iota + 16.0]
  lane_ids = [jnp.full((16,), l, jnp.int32) for l in range(16)]

  def phase1(c, s):
    i_s = c // 8
    j0 = (c % 8) * 4
    iv = jnp.full((16,), i_s).astype(jnp.float32)

    for u in range(NGROUP):
      jv = jnp.full((16,), j0 + u // 2).astype(jnp.float32)
      kv = kvs[u % 2]

      fs = []
      ci = []
      for r in range(3):
        t0, t1, t2, t3 = tv[4 * r:4 * r + 4]
        xs = t0 * jv + t1 * iv + t2 * kv + t3 + 2.0
        x0 = jnp.minimum(xs.astype(jnp.int32), 34)
        x0 = jnp.maximum(x0, 0)
        xd = xs - x0.astype(jnp.float32)
        xi0 = x0 - 2
        xi1 = x0 - 1
        zero = jnp.zeros((16,), jnp.float32)
        f0 = jnp.where((xi0 >= 0) & (xi0 <= 31), 1.0 - xd, zero)
        f1 = jnp.where((xi1 >= 0) & (xi1 <= 31), xd, zero)
        fs.append((f0, f1))
        ci.append((jnp.clip(xi0, 0, 31), jnp.clip(xi1, 0, 31)))

      ax = (ci[0][0] * 1024 + base_row, ci[0][1] * 1024 + base_row)
      by = (ci[1][0] * 32, ci[1][1] * 32)
      cz = ci[2]
      wx, wy, wz = fs
      for xc in range(2):
        for yc in range(2):
          rxy = ax[xc] + by[yc]
          wxy = wx[xc] * wy[yc]
          for zc in range(2):
            cr = (xc << 2) | (yc << 1) | zc
            idx_b[s, cr, pl.ds(u * 16, 16)] = rxy + cz[zc]
            w_b[s, cr, pl.ds(u * 16, 16)] = wxy * wz[zc]

    for cr in range(8):
      pltpu.async_copy(vol_hbm.at[idx_b.at[s, cr]], cbuf.at[s, cr], sems[s])

  def combine(c, s):
    for cr in range(8):
      pltpu.make_async_copy(
          vol_hbm.at[idx_b.at[s, cr]], cbuf.at[s, cr], sems[s]).wait()

    def group_body(g, _):
      wv = [w_b[s, cr, pl.ds(g * 16, 16)] for cr in range(8)]
      for l in range(16):
        v = g * 16 + l
        terms = []
        for cr in range(8):
          wb = wv[cr].at[lane_ids[l]].get(mode="promise_in_bounds")
          terms.append(wb * cbuf[s, cr, v, :])
        t01 = terms[0] + terms[1]
        t23 = terms[2] + terms[3]
        t45 = terms[4] + terms[5]
        t67 = terms[6] + terms[7]
        outb[s, v, :] = (t01 + t23) + (t45 + t67)
      return 0

    lax.fori_loop(0, NGROUP, group_body, 0)
    pltpu.sync_copy(outb.at[s], out_hbm.at[pl.ds(base_row + c * CHUNK, CHUNK)])

  phase1(0, 0)

  def pair_body(c2, _):
    c = c2 * 2
    phase1(c + 1, 1)
    combine(c, 0)

    @pl.when(c2 < NCHUNK // 2 - 1)
    def _():
      phase1(c + 2, 0)

    combine(c + 1, 1)
    return 0

  lax.fori_loop(0, NCHUNK // 2, pair_body, 0)


@jax.jit
def kernel(input_fmap, theta):
  vol = input_fmap.reshape(NVOL * VPV, C)
  tb = lax.bitcast_convert_type(theta, jnp.uint32)
  tb = (tb + jnp.uint32(0x7FFF) + ((tb >> 16) & jnp.uint32(1)))
  tb = tb & jnp.uint32(0xFFFF0000)
  theta_r = lax.bitcast_convert_type(tb, jnp.float32)
  thetab = jnp.broadcast_to(theta_r.reshape(NVOL, 12, 1), (NVOL, 12, 16))

  mesh = plsc.VectorSubcoreMesh(core_axis_name="c", subcore_axis_name="s",
                                num_cores=2, num_subcores=16)
  run = functools.partial(
      pl.kernel,
      mesh=mesh,
      compiler_params=pltpu.CompilerParams(needs_layout_passes=False,
                                           use_tc_tiling_on_sc=False),
      out_type=jax.ShapeDtypeStruct((NVOL * VPV, C), jnp.float32),
      scratch_types=[
          pltpu.VMEM((12, 16), jnp.float32),
          pltpu.VMEM((2, 8, CHUNK), jnp.int32),
          pltpu.VMEM((2, 8, CHUNK), jnp.float32),
          pltpu.VMEM((2, 8, CHUNK, 16), jnp.float32),
          pltpu.VMEM((2, CHUNK, 16), jnp.float32),
          pltpu.SemaphoreType.DMA,
          pltpu.SemaphoreType.DMA,
      ],
  )(_sc_resample)
  out = run(vol, thetab)
  return out.reshape(B, P, H, W, D, C)
